# baseline (speedup 1.0000x reference)
import functools

import jax
import jax.numpy as jnp
from jax import lax
from jax.experimental import pallas as pl
from jax.experimental.pallas import tpu as pltpu
from jax.experimental.pallas import tpu_sc as plsc

VOCAB = 1000000
MAXLEN = 200
EMBED = 64
BATCH = 4096

NC = 2
NS = 16
NW = NC * NS
BW = BATCH // NW
NCHUNK = BW
OROWS = MAXLEN // 2
SUBS = ((0, 96), (96, 104))


_mesh = plsc.VectorSubcoreMesh(core_axis_name="c", subcore_axis_name="s")


@functools.partial(
    pl.kernel,
    mesh=_mesh,
    out_type=jax.ShapeDtypeStruct((BATCH * MAXLEN // 2, 128), jnp.float32),
    scratch_types=[
        pltpu.VMEM((2, MAXLEN), jnp.int32),
        pltpu.VMEM((MAXLEN, EMBED), jnp.float32),
        pltpu.VMEM((MAXLEN, EMBED), jnp.float32),
        pltpu.VMEM((OROWS, 128), jnp.float32),
        pltpu.VMEM((OROWS, 128), jnp.float32),
        pltpu.VMEM((MAXLEN, EMBED), jnp.float32),
        pltpu.SemaphoreType.DMA,
        pltpu.SemaphoreType.DMA,
        pltpu.SemaphoreType.DMA,
        pltpu.SemaphoreType.DMA,
    ],
    compiler_params=pltpu.CompilerParams(use_tc_tiling_on_sc=False),
)
def _embed_kernel(x_hbm, tok_hbm, pos_hbm, out_hbm,
                  idx_v, gbuf0, gbuf1, obuf0, obuf1, pos_v,
                  gsem0, gsem1, osem0, osem1):
    wid = lax.axis_index("s") * NC + lax.axis_index("c")
    batch_base = wid * BW
    gbufs = (gbuf0, gbuf1)
    obufs = (obuf0, obuf1)
    gsems = (gsem0, gsem1)
    osems = (osem0, osem1)

    pltpu.sync_copy(pos_hbm, pos_v)

    def gather_parts(slot):
        parts = []
        for off, size in SUBS:
            parts.append((
                tok_hbm.at[idx_v.at[slot, pl.ds(off, size)]],
                gbufs[slot].at[pl.ds(off, size)],
                gsems[slot],
            ))
        return parts

    def issue_gather(c, slot):
        pltpu.sync_copy(x_hbm.at[pl.ds(batch_base + c, 1)],
                        idx_v.at[pl.ds(slot, 1)])
        for src, dst, sem in gather_parts(slot):
            pltpu.async_copy(src, dst, sem)

    def wait_gather(slot):
        for src, dst, sem in gather_parts(slot):
            pltpu.make_async_copy(src, dst, sem).wait()

    def issue_out(c, slot):
        base = (batch_base + c) * OROWS
        pltpu.async_copy(obufs[slot], out_hbm.at[pl.ds(base, OROWS)],
                         osems[slot])

    def wait_out(slot):
        pltpu.make_async_copy(obufs[slot],
                              out_hbm.at[pl.ds(batch_base * OROWS, OROWS)],
                              osems[slot]).wait()

    issue_gather(0, 0)

    def pair_body(i, carry):
        c0 = 2 * i
        for slot in range(2):
            c = c0 + slot
            nslot = 1 - slot
            nxt = c + 1

            @pl.when(nxt < NCHUNK)
            def _prefetch():
                @pl.when(c >= 1)
                def _reclaim():
                    wait_out(nslot)
                issue_gather(nxt, nslot)

            wait_gather(slot)
            gbuf = gbufs[slot]
            obuf = obufs[slot]

            def pos_add(j, carry2):
                for h in range(2):
                    l = 2 * j + h
                    for q in range(EMBED // 16):
                        sl = pl.ds(q * 16, 16)
                        obuf[j, pl.ds(h * EMBED + q * 16, 16)] = (
                            gbuf[l, sl] + pos_v[l, sl])
                return carry2

            lax.fori_loop(0, OROWS, pos_add, 0)
            issue_out(c, slot)
        return carry

    lax.fori_loop(0, NCHUNK // 2, pair_body, 0)
    wait_out(0)
    wait_out(1)


def kernel(x, token_table, pos_table):
    out = _embed_kernel(x.astype(jnp.int32), token_table, pos_table)
    return out.reshape(BATCH, MAXLEN, EMBED)
